# baseline (speedup 1.0000x reference)
import functools

import jax
import jax.numpy as jnp
from jax import lax
from jax.experimental import pallas as pl
from jax.experimental.pallas import tpu as pltpu
from jax.experimental.pallas import tpu_sc as plsc

ALPHA_C = 1000.0
L = 16


def _rsqrt16(x):
    i = lax.bitcast_convert_type(x, jnp.int32)
    i = jnp.int32(0x5F3759DF) - lax.shift_right_arithmetic(i, 1)
    r = lax.bitcast_convert_type(i, jnp.float32)
    hx = 0.5 * x
    for _ in range(2):
        r = r * (1.5 - hx * r * r)
    return r


def _bf16_hi_to_f32(bits_i32):
    return lax.bitcast_convert_type(
        lax.bitwise_and(bits_i32, jnp.int32(-65536)), jnp.float32)


def _bf16_lo_to_f32(bits_i32):
    return lax.bitcast_convert_type(
        lax.shift_left(bits_i32, 16), jnp.float32)


def _make_sc_kernel(n_nodes_pad, n_edges, n_graphs, chunk):
    NC, NS = 2, 16
    NW = NC * NS
    per_w = n_edges // NW
    n_chunks = per_w // chunk
    nodes_per_tile = n_nodes_pad // NS
    vecs_per_chunk = chunk // L

    mesh = plsc.VectorSubcoreMesh(core_axis_name="c", subcore_axis_name="s")

    @functools.partial(
        pl.kernel,
        out_type=jax.ShapeDtypeStruct((NC * 64,), jnp.float32),
        mesh=mesh,
        compiler_params=pltpu.CompilerParams(
            needs_layout_passes=False, use_tc_tiling_on_sc=False),
        scratch_types=[
            pltpu.VMEM_SHARED((n_nodes_pad,), jnp.int32),
            pltpu.VMEM_SHARED((n_nodes_pad,), jnp.int32),
            pltpu.VMEM_SHARED((NS, n_graphs * L), jnp.float32),
            pltpu.VMEM((nodes_per_tile,), jnp.int32),
            pltpu.VMEM((64,), jnp.int32),
            pltpu.VMEM((chunk,), jnp.int32),
            pltpu.VMEM((chunk,), jnp.int32),
            pltpu.VMEM((chunk * 2,), jnp.float32),
            pltpu.VMEM((chunk,), jnp.int32),
            pltpu.VMEM((chunk,), jnp.int32),
            pltpu.VMEM((chunk,), jnp.int32),
            pltpu.VMEM((chunk,), jnp.int32),
            pltpu.VMEM((n_graphs * L,), jnp.float32),
            pltpu.VMEM((NS, n_graphs * L), jnp.float32),
            pltpu.VMEM((64,), jnp.float32),
        ],
    )
    def sc_kernel(w0_hbm, w1_hbm, src_hbm, dst_hbm, attr_hbm,
                  ptr_hbm, out_hbm,
                  w0_sh, w1_sh, acc_sh,
                  stage_v, ptr_v, sidx_v, didx_v, attr_v,
                  sw0_v, sw1_v, dw0_v, dw1_v,
                  acc_v, accall_v, out_v):
        cid = lax.axis_index("c")
        sid = lax.axis_index("s")
        wid = cid * NS + sid

        node_lo = sid * nodes_per_tile

        pltpu.sync_copy(w0_hbm.at[pl.ds(node_lo, nodes_per_tile)], stage_v)
        pltpu.sync_copy(stage_v, w0_sh.at[pl.ds(node_lo, nodes_per_tile)])

        pltpu.sync_copy(ptr_hbm, ptr_v)
        ptr_vecs = [ptr_v[pl.ds(k * L, L)] for k in range(64 // L)]
        bounds = [ptr_vecs[j // L][j % L] for j in range(1, n_graphs)]
        lane = lax.iota(jnp.int32, L)

        pltpu.sync_copy(w1_hbm.at[pl.ds(node_lo, nodes_per_tile)], stage_v)

        def g_body(k, _):
            n = node_lo + k * L + lane
            cnt = jnp.zeros((L,), jnp.int32)
            for b in bounds:
                cnt = cnt + jnp.where(n >= b, 1, 0).astype(jnp.int32)
            o = k * L
            stage_v[pl.ds(o, L)] = lax.bitwise_or(
                stage_v[pl.ds(o, L)], lax.shift_left(cnt, 16))
            return 0

        lax.fori_loop(0, nodes_per_tile // L, g_body, 0)
        pltpu.sync_copy(stage_v, w1_sh.at[pl.ds(node_lo, nodes_per_tile)])

        zero16 = jnp.zeros((L,), jnp.float32)

        def z_body(i, _):
            acc_v[pl.ds(i * L, L)] = zero16
            return 0

        lax.fori_loop(0, n_graphs, z_body, 0)

        plsc.subcore_barrier()

        edge_base = wid * per_w

        def chunk_body(i, _):
            lo = pl.multiple_of(edge_base + i * chunk, 8)
            pltpu.sync_copy(src_hbm.at[pl.ds(lo, chunk)], sidx_v)
            pltpu.sync_copy(dst_hbm.at[pl.ds(lo, chunk)], didx_v)
            pltpu.sync_copy(attr_hbm.at[pl.ds(lo * 2, chunk * 2)], attr_v)
            pltpu.sync_copy(w0_sh.at[sidx_v], sw0_v)
            pltpu.sync_copy(w1_sh.at[sidx_v], sw1_v)
            pltpu.sync_copy(w0_sh.at[didx_v], dw0_v)
            pltpu.sync_copy(w1_sh.at[didx_v], dw1_v)

            def vec_body(k, _):
                o = k * L
                sw0 = sw0_v[pl.ds(o, L)]
                sw1 = sw1_v[pl.ds(o, L)]
                dw0 = dw0_v[pl.ds(o, L)]
                dw1 = dw1_v[pl.ds(o, L)]
                ddx = _bf16_lo_to_f32(sw0) - _bf16_lo_to_f32(dw0)
                ddy = _bf16_hi_to_f32(sw0) - _bf16_hi_to_f32(dw0)
                ddz = _bf16_lo_to_f32(sw1) - _bf16_lo_to_f32(dw1)
                g = lax.shift_right_logical(sw1, 16)
                d2 = ddx * ddx + ddy * ddy + ddz * ddz + 1e-12
                dist = d2 * _rsqrt16(d2)
                eidx2 = (o + lane) * 2
                r0 = plsc.load_gather(attr_v, [eidx2])
                w = plsc.load_gather(attr_v, [eidx2 + 1])
                diff = dist - r0
                e = (ALPHA_C * w) * (diff * diff)
                plsc.addupdate_scatter(acc_v, [g * L + lane], e)
                return 0

            lax.fori_loop(0, vecs_per_chunk, vec_body, 0)
            return 0

        lax.fori_loop(0, n_chunks, chunk_body, 0)

        pltpu.sync_copy(acc_v, acc_sh.at[sid])
        plsc.subcore_barrier()

        @pl.when(sid == 0)
        def _():
            pltpu.sync_copy(acc_sh, accall_v)
            for k in range(64 // L):
                row = zero16
                for j in range(L):
                    gi = k * L + j
                    if gi >= n_graphs:
                        break
                    tot = accall_v[0, pl.ds(gi * L, L)]
                    for t in range(1, NS):
                        tot = tot + accall_v[t, pl.ds(gi * L, L)]
                    row = jnp.where(lane == j, jnp.sum(tot), row)
                out_v[pl.ds(k * L, L)] = row
            pltpu.sync_copy(out_v, out_hbm.at[pl.ds(cid * 64, 64)])

    return sc_kernel


def kernel(positions, edge_attrs, edge_index, ptr):
    n_nodes = positions.shape[0]
    n_edges = edge_index.shape[1]
    n_graphs = ptr.shape[0] - 1

    n_nodes_pad = ((n_nodes + 127) // 128) * 128
    pad = n_nodes_pad - n_nodes
    pb = lax.bitcast_convert_type(
        positions.astype(jnp.bfloat16), jnp.uint16).astype(jnp.int32)
    w0 = jnp.pad(pb[:, 0] | (pb[:, 1] << 16), (0, pad))
    w1 = jnp.pad(pb[:, 2], (0, pad))
    src = edge_index[0]
    dst = edge_index[1]
    ptr64 = jnp.pad(ptr, (0, 64 - ptr.shape[0]))

    sc = _make_sc_kernel(n_nodes_pad, n_edges, n_graphs, chunk=4000)
    out2 = sc(w0, w1, src, dst, edge_attrs.reshape(-1), ptr64)
    return (out2[:64] + out2[64:])[:n_graphs]


# speedup vs baseline: 10.2909x; 10.2909x over previous
import functools

import jax
import jax.numpy as jnp
from jax import lax
from jax.experimental import pallas as pl
from jax.experimental.pallas import tpu as pltpu
from jax.experimental.pallas import tpu_sc as plsc

ALPHA_C = 1000.0
L = 16


def _rsqrt16(x):
    i = lax.bitcast_convert_type(x, jnp.int32)
    i = jnp.int32(0x5F3759DF) - lax.shift_right_arithmetic(i, 1)
    r = lax.bitcast_convert_type(i, jnp.float32)
    hx = 0.5 * x
    for _ in range(2):
        r = r * (1.5 - hx * r * r)
    return r


def _bf16_hi_to_f32(bits_i32):
    return lax.bitcast_convert_type(
        lax.bitwise_and(bits_i32, jnp.int32(-65536)), jnp.float32)


def _bf16_lo_to_f32(bits_i32):
    return lax.bitcast_convert_type(
        lax.shift_left(bits_i32, 16), jnp.float32)


def _make_sc_kernel(n_nodes_pad, n_edges, n_graphs, chunk):
    NC, NS = 2, 16
    NW = NC * NS
    per_w = n_edges // NW
    n_chunks = per_w // chunk
    nodes_per_tile = n_nodes_pad // NS
    vecs_per_chunk = chunk // L

    mesh = plsc.VectorSubcoreMesh(core_axis_name="c", subcore_axis_name="s")

    @functools.partial(
        pl.kernel,
        out_type=jax.ShapeDtypeStruct((NC * 64,), jnp.float32),
        mesh=mesh,
        compiler_params=pltpu.CompilerParams(
            needs_layout_passes=False, use_tc_tiling_on_sc=False),
        scratch_types=[
            pltpu.VMEM_SHARED((n_nodes_pad,), jnp.int32),
            pltpu.VMEM_SHARED((n_nodes_pad,), jnp.int32),
            pltpu.VMEM_SHARED((NS, n_graphs * L), jnp.float32),
            pltpu.VMEM((nodes_per_tile,), jnp.int32),
            pltpu.VMEM((64,), jnp.int32),
            pltpu.VMEM((chunk,), jnp.int32),
            pltpu.VMEM((chunk,), jnp.int32),
            pltpu.VMEM((chunk,), jnp.float32),
            pltpu.VMEM((chunk,), jnp.float32),
            pltpu.VMEM((chunk,), jnp.int32),
            pltpu.VMEM((chunk,), jnp.int32),
            pltpu.VMEM((chunk,), jnp.int32),
            pltpu.VMEM((chunk,), jnp.int32),
            pltpu.VMEM((n_graphs * L,), jnp.float32),
            pltpu.VMEM((NS, n_graphs * L), jnp.float32),
            pltpu.VMEM((64,), jnp.float32),
        ],
    )
    def sc_kernel(w0_hbm, w1_hbm, src_hbm, dst_hbm, r0_hbm, w_hbm,
                  ptr_hbm, out_hbm,
                  w0_sh, w1_sh, acc_sh,
                  stage_v, ptr_v, sidx_v, didx_v, r0_v, w_v,
                  sw0_v, sw1_v, dw0_v, dw1_v,
                  acc_v, accall_v, out_v):
        cid = lax.axis_index("c")
        sid = lax.axis_index("s")
        wid = cid * NS + sid

        node_lo = sid * nodes_per_tile

        pltpu.sync_copy(w0_hbm.at[pl.ds(node_lo, nodes_per_tile)], stage_v)
        pltpu.sync_copy(stage_v, w0_sh.at[pl.ds(node_lo, nodes_per_tile)])

        pltpu.sync_copy(ptr_hbm, ptr_v)
        ptr_vecs = [ptr_v[pl.ds(k * L, L)] for k in range(64 // L)]
        bounds = [ptr_vecs[j // L][j % L] for j in range(1, n_graphs)]
        lane = lax.iota(jnp.int32, L)

        pltpu.sync_copy(w1_hbm.at[pl.ds(node_lo, nodes_per_tile)], stage_v)

        def g_body(k, _):
            n = node_lo + k * L + lane
            cnt = jnp.zeros((L,), jnp.int32)
            for b in bounds:
                cnt = cnt + jnp.where(n >= b, 1, 0).astype(jnp.int32)
            o = k * L
            stage_v[pl.ds(o, L)] = lax.bitwise_or(
                stage_v[pl.ds(o, L)], lax.shift_left(cnt, 16))
            return 0

        lax.fori_loop(0, nodes_per_tile // L, g_body, 0)
        pltpu.sync_copy(stage_v, w1_sh.at[pl.ds(node_lo, nodes_per_tile)])

        zero16 = jnp.zeros((L,), jnp.float32)

        def z_body(i, _):
            acc_v[pl.ds(i * L, L)] = zero16
            return 0

        lax.fori_loop(0, n_graphs, z_body, 0)

        plsc.subcore_barrier()

        edge_base = wid * per_w

        def chunk_body(i, _):
            lo = pl.multiple_of(edge_base + i * chunk, 8)
            pltpu.sync_copy(src_hbm.at[pl.ds(lo, chunk)], sidx_v)
            pltpu.sync_copy(dst_hbm.at[pl.ds(lo, chunk)], didx_v)
            pltpu.sync_copy(r0_hbm.at[pl.ds(lo, chunk)], r0_v)
            pltpu.sync_copy(w_hbm.at[pl.ds(lo, chunk)], w_v)
            pltpu.sync_copy(w0_sh.at[sidx_v], sw0_v)
            pltpu.sync_copy(w1_sh.at[sidx_v], sw1_v)
            pltpu.sync_copy(w0_sh.at[didx_v], dw0_v)
            pltpu.sync_copy(w1_sh.at[didx_v], dw1_v)

            def vec_body(k, _):
                o = k * L
                sw0 = sw0_v[pl.ds(o, L)]
                sw1 = sw1_v[pl.ds(o, L)]
                dw0 = dw0_v[pl.ds(o, L)]
                dw1 = dw1_v[pl.ds(o, L)]
                ddx = _bf16_lo_to_f32(sw0) - _bf16_lo_to_f32(dw0)
                ddy = _bf16_hi_to_f32(sw0) - _bf16_hi_to_f32(dw0)
                ddz = _bf16_lo_to_f32(sw1) - _bf16_lo_to_f32(dw1)
                g = lax.shift_right_logical(sw1, 16)
                d2 = ddx * ddx + ddy * ddy + ddz * ddz + 1e-12
                dist = d2 * _rsqrt16(d2)
                r0 = r0_v[pl.ds(o, L)]
                w = w_v[pl.ds(o, L)]
                diff = dist - r0
                e = (ALPHA_C * w) * (diff * diff)
                plsc.addupdate_scatter(acc_v, [g * L + lane], e)
                return 0

            lax.fori_loop(0, vecs_per_chunk, vec_body, 0)
            return 0

        lax.fori_loop(0, n_chunks, chunk_body, 0)

        pltpu.sync_copy(acc_v, acc_sh.at[sid])
        plsc.subcore_barrier()

        @pl.when(sid == 0)
        def _():
            pltpu.sync_copy(acc_sh, accall_v)
            for k in range(64 // L):
                row = zero16
                for j in range(L):
                    gi = k * L + j
                    if gi >= n_graphs:
                        break
                    tot = accall_v[0, pl.ds(gi * L, L)]
                    for t in range(1, NS):
                        tot = tot + accall_v[t, pl.ds(gi * L, L)]
                    row = jnp.where(lane == j, jnp.sum(tot), row)
                out_v[pl.ds(k * L, L)] = row
            pltpu.sync_copy(out_v, out_hbm.at[pl.ds(cid * 64, 64)])

    return sc_kernel


def kernel(positions, edge_attrs, edge_index, ptr):
    n_nodes = positions.shape[0]
    n_edges = edge_index.shape[1]
    n_graphs = ptr.shape[0] - 1

    n_nodes_pad = ((n_nodes + 127) // 128) * 128
    pad = n_nodes_pad - n_nodes
    pb = lax.bitcast_convert_type(
        positions.astype(jnp.bfloat16), jnp.uint16).astype(jnp.int32)
    w0 = jnp.pad(pb[:, 0] | (pb[:, 1] << 16), (0, pad))
    w1 = jnp.pad(pb[:, 2], (0, pad))
    src = edge_index[0]
    dst = edge_index[1]
    ptr64 = jnp.pad(ptr, (0, 64 - ptr.shape[0]))

    r0 = edge_attrs[:, 0]
    w = edge_attrs[:, 1]
    sc = _make_sc_kernel(n_nodes_pad, n_edges, n_graphs, chunk=4000)
    out2 = sc(w0, w1, src, dst, r0, w, ptr64)
    return (out2[:64] + out2[64:])[:n_graphs]


# speedup vs baseline: 16.4966x; 1.6030x over previous
import functools

import jax
import jax.numpy as jnp
from jax import lax
from jax.experimental import pallas as pl
from jax.experimental.pallas import tpu as pltpu
from jax.experimental.pallas import tpu_sc as plsc

ALPHA_C = 1000.0
L = 16


def _rsqrt16(x):
    i = lax.bitcast_convert_type(x, jnp.int32)
    i = jnp.int32(0x5F3759DF) - lax.shift_right_arithmetic(i, 1)
    r = lax.bitcast_convert_type(i, jnp.float32)
    hx = 0.5 * x
    for _ in range(2):
        r = r * (1.5 - hx * r * r)
    return r


def _bf16_hi_to_f32(bits_i32):
    return lax.bitcast_convert_type(
        lax.bitwise_and(bits_i32, jnp.int32(-65536)), jnp.float32)


def _bf16_lo_to_f32(bits_i32):
    return lax.bitcast_convert_type(
        lax.shift_left(bits_i32, 16), jnp.float32)


def _make_sc_kernel(n_nodes_pad, n_edges, n_graphs, chunk):
    NC, NS = 2, 16
    NW = NC * NS
    per_w = n_edges // NW
    n_chunks = per_w // chunk
    nodes_per_tile = n_nodes_pad // NS
    vecs_per_chunk = chunk // L

    mesh = plsc.VectorSubcoreMesh(core_axis_name="c", subcore_axis_name="s")

    @functools.partial(
        pl.kernel,
        out_type=jax.ShapeDtypeStruct((NC * 64,), jnp.float32),
        mesh=mesh,
        compiler_params=pltpu.CompilerParams(
            needs_layout_passes=False, use_tc_tiling_on_sc=False),
        scratch_types=[
            pltpu.VMEM_SHARED((n_nodes_pad,), jnp.int32),
            pltpu.VMEM_SHARED((n_nodes_pad,), jnp.int32),
            pltpu.VMEM_SHARED((NS, n_graphs * L), jnp.float32),
            pltpu.VMEM((nodes_per_tile,), jnp.int32),
            pltpu.VMEM((64,), jnp.int32),
            [[pltpu.VMEM((chunk,), jnp.int32),
              pltpu.VMEM((chunk,), jnp.int32),
              pltpu.VMEM((chunk,), jnp.float32),
              pltpu.VMEM((chunk,), jnp.float32),
              pltpu.VMEM((chunk,), jnp.int32),
              pltpu.VMEM((chunk,), jnp.int32),
              pltpu.VMEM((chunk,), jnp.int32),
              pltpu.VMEM((chunk,), jnp.int32)] for _ in range(2)],
            [pltpu.SemaphoreType.DMA for _ in range(4)],
            pltpu.VMEM((n_graphs * L,), jnp.float32),
            pltpu.VMEM((NS, n_graphs * L), jnp.float32),
            pltpu.VMEM((64,), jnp.float32),
        ],
    )
    def sc_kernel(w0_hbm, w1_hbm, src_hbm, dst_hbm, r0_hbm, w_hbm,
                  ptr_hbm, out_hbm,
                  w0_sh, w1_sh, acc_sh,
                  stage_v, ptr_v, bufs2, sems,
                  acc_v, accall_v, out_v):
        cid = lax.axis_index("c")
        sid = lax.axis_index("s")
        wid = cid * NS + sid

        node_lo = sid * nodes_per_tile

        pltpu.sync_copy(w0_hbm.at[pl.ds(node_lo, nodes_per_tile)], stage_v)
        pltpu.sync_copy(stage_v, w0_sh.at[pl.ds(node_lo, nodes_per_tile)])

        pltpu.sync_copy(ptr_hbm, ptr_v)
        ptr_vecs = [ptr_v[pl.ds(k * L, L)] for k in range(64 // L)]
        bounds = [ptr_vecs[j // L][j % L] for j in range(1, n_graphs)]
        lane = lax.iota(jnp.int32, L)

        pltpu.sync_copy(w1_hbm.at[pl.ds(node_lo, nodes_per_tile)], stage_v)

        def g_body(k, _):
            n = node_lo + k * L + lane
            cnt = jnp.zeros((L,), jnp.int32)
            for b in bounds:
                cnt = cnt + jnp.where(n >= b, 1, 0).astype(jnp.int32)
            o = k * L
            stage_v[pl.ds(o, L)] = lax.bitwise_or(
                stage_v[pl.ds(o, L)], lax.shift_left(cnt, 16))
            return 0

        lax.fori_loop(0, nodes_per_tile // L, g_body, 0)
        pltpu.sync_copy(stage_v, w1_sh.at[pl.ds(node_lo, nodes_per_tile)])

        zero16 = jnp.zeros((L,), jnp.float32)

        def z_body(i, _):
            acc_v[pl.ds(i * L, L)] = zero16
            return 0

        lax.fori_loop(0, n_graphs, z_body, 0)

        plsc.subcore_barrier()

        edge_base = wid * per_w
        semL_a, semG_a, semL_b, semG_b = sems

        def linear_descs(bufs, sem, lo, make):
            f = pltpu.make_async_copy if make else pltpu.async_copy
            return [
                f(src_hbm.at[pl.ds(lo, chunk)], bufs[0], sem),
                f(dst_hbm.at[pl.ds(lo, chunk)], bufs[1], sem),
                f(r0_hbm.at[pl.ds(lo, chunk)], bufs[2], sem),
                f(w_hbm.at[pl.ds(lo, chunk)], bufs[3], sem),
            ]

        def gather_descs(bufs, sem, make):
            f = pltpu.make_async_copy if make else pltpu.async_copy
            return [
                f(w0_sh.at[bufs[0]], bufs[4], sem),
                f(w1_sh.at[bufs[0]], bufs[5], sem),
                f(w0_sh.at[bufs[1]], bufs[6], sem),
                f(w1_sh.at[bufs[1]], bufs[7], sem),
            ]

        def fire_chunk(bufs, semL, semG, lo):
            linear_descs(bufs, semL, lo, False)
            for d in linear_descs(bufs, semL, lo, True):
                d.wait()
            gather_descs(bufs, semG, False)

        def drain_gathers(bufs, semG):
            for d in gather_descs(bufs, semG, True):
                d.wait()

        def compute(bufs):
            r0_v, w_v = bufs[2], bufs[3]
            sw0_v, sw1_v, dw0_v, dw1_v = bufs[4], bufs[5], bufs[6], bufs[7]

            def vec_body(k, _):
                o = k * L
                sw0 = sw0_v[pl.ds(o, L)]
                sw1 = sw1_v[pl.ds(o, L)]
                dw0 = dw0_v[pl.ds(o, L)]
                dw1 = dw1_v[pl.ds(o, L)]
                ddx = _bf16_lo_to_f32(sw0) - _bf16_lo_to_f32(dw0)
                ddy = _bf16_hi_to_f32(sw0) - _bf16_hi_to_f32(dw0)
                ddz = _bf16_lo_to_f32(sw1) - _bf16_lo_to_f32(dw1)
                g = lax.shift_right_logical(sw1, 16)
                d2 = ddx * ddx + ddy * ddy + ddz * ddz + 1e-12
                dist = d2 * _rsqrt16(d2)
                r0 = r0_v[pl.ds(o, L)]
                w = w_v[pl.ds(o, L)]
                diff = dist - r0
                e = (ALPHA_C * w) * (diff * diff)
                plsc.addupdate_scatter(acc_v, [g * L + lane], e)
                return 0

            lax.fori_loop(0, vecs_per_chunk, vec_body, 0)

        bufs_a, bufs_b = bufs2
        fire_chunk(bufs_a, semL_a, semG_a, pl.multiple_of(edge_base, 8))
        nb = n_chunks // 2

        def pipe_body(j, _):
            lo_b = pl.multiple_of(edge_base + (2 * j + 1) * chunk, 8)
            fire_chunk(bufs_b, semL_b, semG_b, lo_b)
            drain_gathers(bufs_a, semG_a)
            compute(bufs_a)

            @pl.when(j < nb - 1)
            def _():
                lo_a = pl.multiple_of(edge_base + (2 * j + 2) * chunk, 8)
                fire_chunk(bufs_a, semL_a, semG_a, lo_a)

            drain_gathers(bufs_b, semG_b)
            compute(bufs_b)
            return 0

        lax.fori_loop(0, nb, pipe_body, 0)

        pltpu.sync_copy(acc_v, acc_sh.at[sid])
        plsc.subcore_barrier()

        @pl.when(sid == 0)
        def _():
            pltpu.sync_copy(acc_sh, accall_v)
            for k in range(64 // L):
                row = zero16
                for j in range(L):
                    gi = k * L + j
                    if gi >= n_graphs:
                        break
                    tot = accall_v[0, pl.ds(gi * L, L)]
                    for t in range(1, NS):
                        tot = tot + accall_v[t, pl.ds(gi * L, L)]
                    row = jnp.where(lane == j, jnp.sum(tot), row)
                out_v[pl.ds(k * L, L)] = row
            pltpu.sync_copy(out_v, out_hbm.at[pl.ds(cid * 64, 64)])

    return sc_kernel


def kernel(positions, edge_attrs, edge_index, ptr):
    n_nodes = positions.shape[0]
    n_edges = edge_index.shape[1]
    n_graphs = ptr.shape[0] - 1

    n_nodes_pad = ((n_nodes + 127) // 128) * 128
    pad = n_nodes_pad - n_nodes
    pb = lax.bitcast_convert_type(
        positions.astype(jnp.bfloat16), jnp.uint16).astype(jnp.int32)
    w0 = jnp.pad(pb[:, 0] | (pb[:, 1] << 16), (0, pad))
    w1 = jnp.pad(pb[:, 2], (0, pad))
    src = edge_index[0]
    dst = edge_index[1]
    ptr64 = jnp.pad(ptr, (0, 64 - ptr.shape[0]))

    r0 = edge_attrs[:, 0]
    w = edge_attrs[:, 1]
    sc = _make_sc_kernel(n_nodes_pad, n_edges, n_graphs, chunk=4000)
    out2 = sc(w0, w1, src, dst, r0, w, ptr64)
    return (out2[:64] + out2[64:])[:n_graphs]
